# baseline (speedup 1.0000x reference)
import functools

import jax
import jax.numpy as jnp
from jax import lax
from jax.experimental import pallas as pl
from jax.experimental.pallas import tpu as pltpu
from jax.experimental.pallas import tpu_sc as plsc

N = 10000
E = 160000
DIN = 256
DOUT = 512
DH = DIN // 2

NC = 2
NS = 16
L = 16

EB = 96
NB = 108
NP = NB // 2
ET = NB * EB
EPAD = ET * NS
NPAD = 10240
RPT = NPAD // NS

_GDN = lax.GatherDimensionNumbers(
    offset_dims=(), collapsed_slice_dims=(0,), start_index_map=(0,))


def _lane_bcast(v16, j):
    idx = jnp.full((L, 1), j, jnp.int32)
    return lax.gather(v16, idx, _GDN, slice_sizes=(1,),
                      mode=lax.GatherScatterMode.PROMISE_IN_BOUNDS)


def _spmm_body(xcat, sd4, w3, zrows, agg_out,
               sd_all, w20, w21, rows, gidx0, gidx1, sidx0, sidx1, aggsh,
               gsem0, gsem1, ssem0, ssem1, wsem0, wsem1):
    c = lax.axis_index("c")
    s = lax.axis_index("s")

    pltpu.sync_copy(sd4.at[c, s], sd_all)
    pltpu.sync_copy(zrows, aggsh.at[pl.ds(s * RPT, RPT)])
    pltpu.sync_copy(zrows.at[pl.ds(0, EB)], rows.at[1])
    plsc.subcore_barrier()

    def unpack(b, gi, si):
        for t in range(EB // L):
            wd = sd_all[b, pl.ds(t * L, L)]
            gi[pl.ds(t * L, L)] = jnp.bitwise_and(wd, 0xFFFF)
            si[pl.ds(t * L, L)] = jnp.right_shift(wd, 16)

    def scale(buf, wv):
        for g in range(EB // L):
            w16 = wv[pl.ds(g * L, L)]
            for j in range(L):
                wb = _lane_bcast(w16, j)
                e = g * L + j
                for k in range(DH // L):
                    rows[buf, e, pl.ds(k * L, L)] = (
                        rows[buf, e, pl.ds(k * L, L)] * wb)

    def wload(b, wv, sem):
        return pltpu.async_copy(w3.at[s, b], wv, sem)

    def wwait(b, wv, sem):
        pltpu.make_async_copy(w3.at[s, b], wv, sem).wait()

    EH = EB // 4

    def gather(gi, buf, sem):
        for q in range(4):
            pltpu.async_copy(xcat.at[gi.at[pl.ds(q * EH, EH)]],
                             rows.at[buf, pl.ds(q * EH, EH)], sem)

    def gwait(gi, buf, sem):
        for q in range(4):
            pltpu.make_async_copy(xcat.at[gi.at[pl.ds(q * EH, EH)]],
                                  rows.at[buf, pl.ds(q * EH, EH)],
                                  sem).wait()

    def scatter(buf, si, sem):
        return pltpu.async_copy(rows.at[buf], aggsh.at[si], sem, add=True)

    def swait(buf, si, sem):
        pltpu.make_async_copy(rows.at[buf], aggsh.at[si], sem).wait()

    def perm(b):
        bb = b + c * (NB // 2)
        return jnp.where(bb >= NB, bb - NB, bb)

    unpack(perm(0), gidx0, sidx0)
    scatter(1, sidx0, ssem1)
    gather(gidx0, 0, gsem0)
    wload(perm(0), w20, wsem0)
    wload(perm(1), w21, wsem1)

    def pair(p, _):
        b0 = 2 * p
        b1 = b0 + 1
        swait(1, sidx1, ssem1)
        unpack(perm(b1), gidx1, sidx1)
        gather(gidx1, 1, gsem1)
        gwait(gidx0, 0, gsem0)
        wwait(perm(b0), w20, wsem0)
        scale(0, w20)
        scatter(0, sidx0, ssem0)
        gwait(gidx1, 1, gsem1)
        wwait(perm(b1), w21, wsem1)
        scale(1, w21)
        scatter(1, sidx1, ssem1)
        swait(0, sidx0, ssem0)
        unpack(perm(b0 + 2), gidx0, sidx0)
        gather(gidx0, 0, gsem0)
        wload(perm(b0 + 2), w20, wsem0)
        wload(perm(b1 + 2), w21, wsem1)
        return 0

    lax.fori_loop(0, NP, pair, 0)
    swait(1, sidx1, ssem1)
    gwait(gidx0, 0, gsem0)
    wwait(perm(0), w20, wsem0)
    wwait(perm(1), w21, wsem1)
    plsc.subcore_barrier()
    pltpu.sync_copy(aggsh.at[pl.ds(s * RPT, RPT)],
                    agg_out.at[pl.ds(c * NPAD + s * RPT, RPT)])


_spmm = functools.partial(
    pl.kernel,
    out_type=jax.ShapeDtypeStruct((NC * NPAD, DH), jnp.float32),
    mesh=plsc.VectorSubcoreMesh(core_axis_name="c", subcore_axis_name="s"),
    scratch_types=[
        pltpu.VMEM((NB + 2, EB), jnp.int32),
        pltpu.VMEM((EB,), jnp.float32),
        pltpu.VMEM((EB,), jnp.float32),
        pltpu.VMEM((2, EB, DH), jnp.float32),
        pltpu.VMEM((EB,), jnp.int32),
        pltpu.VMEM((EB,), jnp.int32),
        pltpu.VMEM((EB,), jnp.int32),
        pltpu.VMEM((EB,), jnp.int32),
        pltpu.VMEM_SHARED((NPAD, DH), jnp.float32),
        pltpu.SemaphoreType.DMA,
        pltpu.SemaphoreType.DMA,
        pltpu.SemaphoreType.DMA,
        pltpu.SemaphoreType.DMA,
        pltpu.SemaphoreType.DMA,
        pltpu.SemaphoreType.DMA,
    ],
)(_spmm_body)


def _mm_body(a_ref, w_ref, alpha_ref, o_ref):
    a = a_ref[...]
    acc = jnp.dot(a[0], w_ref[0:DH, :], preferred_element_type=jnp.float32)
    acc = acc + jnp.dot(a[1], w_ref[DH:DIN, :],
                        preferred_element_type=jnp.float32)
    al = alpha_ref[0]
    o_ref[...] = jnp.maximum(acc, 0.0) + al * jnp.minimum(acc, 0.0)


_R = 1000


def _linear_prelu(agg3, W, alpha1):
    return pl.pallas_call(
        _mm_body,
        grid=(N // _R,),
        in_specs=[
            pl.BlockSpec((2, _R, DH), lambda i: (0, i, 0)),
            pl.BlockSpec((DIN, DOUT), lambda i: (0, 0)),
            pl.BlockSpec(memory_space=pltpu.SMEM),
        ],
        out_specs=pl.BlockSpec((_R, DOUT), lambda i: (i, 0)),
        out_shape=jax.ShapeDtypeStruct((N, DOUT), jnp.float32),
    )(agg3, W, alpha1)


def kernel(x, edge_index, edge_weight, W, alpha):
    src = edge_index[1].astype(jnp.int32)
    dst = edge_index[0].astype(jnp.int32)
    w = edge_weight.astype(jnp.float32)
    pad = EPAD - E
    src_p = jnp.concatenate([src, jnp.zeros((pad,), jnp.int32)])
    dst_p = jnp.concatenate([dst, jnp.zeros((pad,), jnp.int32)])
    w_p = jnp.concatenate([w, jnp.zeros((pad,), jnp.float32)])
    dhi = jnp.left_shift(dst_p, 16)
    sd = jnp.stack([dhi | src_p, dhi | (src_p + N)])
    sd4 = jnp.pad(sd.reshape(NC, NS, NB, EB), ((0, 0), (0, 0), (0, 2),
                                               (0, 0)))
    w3 = jnp.pad(w_p.reshape(NS, NB, EB), ((0, 0), (0, 2), (0, 0)))
    zrows = jnp.zeros((RPT, DH), jnp.float32)
    xcat = jnp.concatenate([x[:, :DH], x[:, DH:]], axis=0)
    agg = _spmm(xcat, sd4, w3, zrows)
    agg3 = agg.reshape(NC, NPAD, DH)
    return _linear_prelu(agg3, W, alpha.reshape(1))


# speedup vs baseline: 1.4451x; 1.4451x over previous
import functools

import jax
import jax.numpy as jnp
from jax import lax
from jax.experimental import pallas as pl
from jax.experimental.pallas import tpu as pltpu
from jax.experimental.pallas import tpu_sc as plsc

N = 10000
E = 160000
DIN = 256
DOUT = 512
DH = DIN // 2

NC = 2
NS = 16
L = 16

EB = 96
NB = 106
NP = NB // 2
ET = NB * EB
EPAD = ET * NS
NPAD = 10240
RPT = NPAD // NS

_GDN = lax.GatherDimensionNumbers(
    offset_dims=(), collapsed_slice_dims=(0,), start_index_map=(0,))


def _lane_bcast(v16, j):
    idx = jnp.full((L, 1), j, jnp.int32)
    return lax.gather(v16, idx, _GDN, slice_sizes=(1,),
                      mode=lax.GatherScatterMode.PROMISE_IN_BOUNDS)


def _spmm_body(xcat, sd4, w3, zrows, agg_out,
               sd_all, w20, w21, rows, gidx0, gidx1, sidx0, sidx1, aggsh,
               gsem0, gsem1, ssem0, ssem1, wsem0, wsem1):
    c = lax.axis_index("c")
    s = lax.axis_index("s")

    pltpu.sync_copy(sd4.at[c, s], sd_all)
    pltpu.sync_copy(zrows, aggsh.at[pl.ds(s * RPT, RPT)])
    pltpu.sync_copy(zrows.at[pl.ds(0, EB)], rows.at[1])
    plsc.subcore_barrier()

    def unpack(b, gi, si):
        for t in range(EB // L):
            wd = sd_all[b, pl.ds(t * L, L)]
            gi[pl.ds(t * L, L)] = jnp.bitwise_and(wd, 0xFFFF)
            si[pl.ds(t * L, L)] = jnp.right_shift(wd, 16)

    def scale(buf, wv):
        for g in range(EB // L):
            w16 = wv[pl.ds(g * L, L)]
            for j in range(L):
                wb = _lane_bcast(w16, j)
                e = g * L + j
                for k in range(DH // L):
                    rows[buf, e, pl.ds(k * L, L)] = (
                        rows[buf, e, pl.ds(k * L, L)] * wb)

    def wload(b, wv, sem):
        return pltpu.async_copy(w3.at[s, b], wv, sem)

    def wwait(b, wv, sem):
        pltpu.make_async_copy(w3.at[s, b], wv, sem).wait()

    EH = EB // 2

    def gather(gi, buf, sem):
        pltpu.async_copy(xcat.at[gi.at[pl.ds(0, EH)]],
                         rows.at[buf, pl.ds(0, EH)], sem)
        pltpu.async_copy(xcat.at[gi.at[pl.ds(EH, EH)]],
                         rows.at[buf, pl.ds(EH, EH)], sem)

    def gwait(gi, buf, sem):
        pltpu.make_async_copy(xcat.at[gi.at[pl.ds(0, EH)]],
                              rows.at[buf, pl.ds(0, EH)], sem).wait()
        pltpu.make_async_copy(xcat.at[gi.at[pl.ds(EH, EH)]],
                              rows.at[buf, pl.ds(EH, EH)], sem).wait()

    def scatter(buf, si, sem):
        return pltpu.async_copy(rows.at[buf], aggsh.at[si], sem, add=True)

    def swait(buf, si, sem):
        pltpu.make_async_copy(rows.at[buf], aggsh.at[si], sem).wait()

    def perm(b):
        bb = b + c * (NB // 2)
        return jnp.where(bb >= NB, bb - NB, bb)

    unpack(perm(0), gidx0, sidx0)
    scatter(1, sidx0, ssem1)
    gather(gidx0, 0, gsem0)
    wload(perm(0), w20, wsem0)
    wload(perm(1), w21, wsem1)

    def pair(p, _):
        b0 = 2 * p
        b1 = b0 + 1
        swait(1, sidx1, ssem1)
        unpack(perm(b1), gidx1, sidx1)
        gather(gidx1, 1, gsem1)
        gwait(gidx0, 0, gsem0)
        wwait(perm(b0), w20, wsem0)
        scale(0, w20)
        scatter(0, sidx0, ssem0)
        gwait(gidx1, 1, gsem1)
        wwait(perm(b1), w21, wsem1)
        scale(1, w21)
        scatter(1, sidx1, ssem1)
        swait(0, sidx0, ssem0)
        unpack(perm(b0 + 2), gidx0, sidx0)
        gather(gidx0, 0, gsem0)
        wload(perm(b0 + 2), w20, wsem0)
        wload(perm(b1 + 2), w21, wsem1)
        return 0

    lax.fori_loop(0, NP, pair, 0)
    swait(1, sidx1, ssem1)
    gwait(gidx0, 0, gsem0)
    wwait(perm(0), w20, wsem0)
    wwait(perm(1), w21, wsem1)
    plsc.subcore_barrier()
    pltpu.sync_copy(aggsh.at[pl.ds(s * RPT, RPT)],
                    agg_out.at[pl.ds(c * NPAD + s * RPT, RPT)])


_spmm = functools.partial(
    pl.kernel,
    out_type=jax.ShapeDtypeStruct((NC * NPAD, DH), jnp.float32),
    mesh=plsc.VectorSubcoreMesh(core_axis_name="c", subcore_axis_name="s"),
    scratch_types=[
        pltpu.VMEM((NB + 2, EB), jnp.int32),
        pltpu.VMEM((EB,), jnp.float32),
        pltpu.VMEM((EB,), jnp.float32),
        pltpu.VMEM((2, EB, DH), jnp.float32),
        pltpu.VMEM((EB,), jnp.int32),
        pltpu.VMEM((EB,), jnp.int32),
        pltpu.VMEM((EB,), jnp.int32),
        pltpu.VMEM((EB,), jnp.int32),
        pltpu.VMEM_SHARED((NPAD, DH), jnp.float32),
        pltpu.SemaphoreType.DMA,
        pltpu.SemaphoreType.DMA,
        pltpu.SemaphoreType.DMA,
        pltpu.SemaphoreType.DMA,
        pltpu.SemaphoreType.DMA,
        pltpu.SemaphoreType.DMA,
    ],
)(_spmm_body)


def _mm_body(a_ref, w_ref, alpha_ref, o_ref):
    a = a_ref[...]
    acc = jnp.dot(a[0], w_ref[0:DH, :], preferred_element_type=jnp.float32)
    acc = acc + jnp.dot(a[1], w_ref[DH:DIN, :],
                        preferred_element_type=jnp.float32)
    al = alpha_ref[0]
    o_ref[...] = jnp.maximum(acc, 0.0) + al * jnp.minimum(acc, 0.0)


_R = 1000


def _linear_prelu(agg3, W, alpha1):
    return pl.pallas_call(
        _mm_body,
        grid=(N // _R,),
        in_specs=[
            pl.BlockSpec((2, _R, DH), lambda i: (0, i, 0)),
            pl.BlockSpec((DIN, DOUT), lambda i: (0, 0)),
            pl.BlockSpec(memory_space=pltpu.SMEM),
        ],
        out_specs=pl.BlockSpec((_R, DOUT), lambda i: (i, 0)),
        out_shape=jax.ShapeDtypeStruct((N, DOUT), jnp.float32),
    )(agg3, W, alpha1)


def kernel(x, edge_index, edge_weight, W, alpha):
    src = edge_index[1].astype(jnp.int32)
    dst = edge_index[0].astype(jnp.int32)
    w = edge_weight.astype(jnp.float32)
    pad = EPAD - E
    src_p = jnp.concatenate([src, jnp.zeros((pad,), jnp.int32)])
    dst_p = jnp.concatenate([dst, jnp.zeros((pad,), jnp.int32)])
    w_p = jnp.concatenate([w, jnp.zeros((pad,), jnp.float32)])
    dhi = jnp.left_shift(dst_p, 16)
    sd = jnp.stack([dhi | src_p, dhi | (src_p + N)])
    sd4 = jnp.pad(sd.reshape(NC, NS, NB, EB), ((0, 0), (0, 0), (0, 2),
                                               (0, 0)))
    w3 = jnp.pad(w_p.reshape(NS, NB, EB), ((0, 0), (0, 2), (0, 0)))
    zrows = jnp.zeros((RPT, DH), jnp.float32)
    xcat = jnp.concatenate([x[:, :DH], x[:, DH:]], axis=0)
    agg = _spmm(xcat, sd4, w3, zrows)
    agg3 = agg.reshape(NC, NPAD, DH)
    return _linear_prelu(agg3, W, alpha.reshape(1))


# speedup vs baseline: 1.4796x; 1.0239x over previous
import functools

import jax
import jax.numpy as jnp
from jax import lax
from jax.experimental import pallas as pl
from jax.experimental.pallas import tpu as pltpu
from jax.experimental.pallas import tpu_sc as plsc

N = 10000
E = 160000
DIN = 256
DOUT = 512
DH = DIN // 2

NC = 2
NS = 16
L = 16

EB = 96
NB = 106
NP = NB // 2
ET = NB * EB
EPAD = ET * NS
NPAD = 10240
RPT = NPAD // NS

_GDN = lax.GatherDimensionNumbers(
    offset_dims=(), collapsed_slice_dims=(0,), start_index_map=(0,))


def _lane_bcast(v16, j):
    idx = jnp.full((L, 1), j, jnp.int32)
    return lax.gather(v16, idx, _GDN, slice_sizes=(1,),
                      mode=lax.GatherScatterMode.PROMISE_IN_BOUNDS)


def _spmm_body(xcat, sd4, w3, zrows, agg_out,
               sd_all, w20, w21, rows, gidx0, gidx1, sidx0, sidx1, aggsh,
               gsem0, gsem1, ssem0, ssem1, wsem0, wsem1):
    c = lax.axis_index("c")
    s = lax.axis_index("s")

    pltpu.sync_copy(sd4.at[c, s], sd_all)
    pltpu.sync_copy(zrows, aggsh.at[pl.ds(s * RPT, RPT)])
    pltpu.sync_copy(zrows.at[pl.ds(0, EB)], rows.at[1])
    plsc.subcore_barrier()

    def unpack(b, gi, si):
        for t in range(EB // L):
            wd = sd_all[b, pl.ds(t * L, L)]
            gi[pl.ds(t * L, L)] = jnp.bitwise_and(wd, 0xFFFF)
            si[pl.ds(t * L, L)] = jnp.right_shift(wd, 16)

    def scale(buf, wv):
        for g in range(EB // L):
            w16 = wv[pl.ds(g * L, L)]
            for j in range(L):
                wb = _lane_bcast(w16, j)
                e = g * L + j
                for k in range(DH // L):
                    rows[buf, e, pl.ds(k * L, L)] = (
                        rows[buf, e, pl.ds(k * L, L)] * wb)

    def wload(b, wv, sem):
        return pltpu.async_copy(w3.at[s, b], wv, sem)

    def wwait(b, wv, sem):
        pltpu.make_async_copy(w3.at[s, b], wv, sem).wait()

    EH = EB // 4

    def gather(gi, buf, sem):
        for q in range(4):
            pltpu.async_copy(xcat.at[gi.at[pl.ds(q * EH, EH)]],
                             rows.at[buf, pl.ds(q * EH, EH)], sem)

    def gwait(gi, buf, sem):
        for q in range(4):
            pltpu.make_async_copy(xcat.at[gi.at[pl.ds(q * EH, EH)]],
                                  rows.at[buf, pl.ds(q * EH, EH)],
                                  sem).wait()

    def scatter(buf, si, sem):
        return pltpu.async_copy(rows.at[buf], aggsh.at[si], sem, add=True)

    def swait(buf, si, sem):
        pltpu.make_async_copy(rows.at[buf], aggsh.at[si], sem).wait()

    def perm(b):
        bb = b + c * (NB // 2)
        return jnp.where(bb >= NB, bb - NB, bb)

    unpack(perm(0), gidx0, sidx0)
    scatter(1, sidx0, ssem1)
    gather(gidx0, 0, gsem0)
    wload(perm(0), w20, wsem0)
    wload(perm(1), w21, wsem1)

    def pair(p, _):
        b0 = 2 * p
        b1 = b0 + 1
        swait(1, sidx1, ssem1)
        unpack(perm(b1), gidx1, sidx1)
        gather(gidx1, 1, gsem1)
        gwait(gidx0, 0, gsem0)
        wwait(perm(b0), w20, wsem0)
        scale(0, w20)
        scatter(0, sidx0, ssem0)
        gwait(gidx1, 1, gsem1)
        wwait(perm(b1), w21, wsem1)
        scale(1, w21)
        scatter(1, sidx1, ssem1)
        swait(0, sidx0, ssem0)
        unpack(perm(b0 + 2), gidx0, sidx0)
        gather(gidx0, 0, gsem0)
        wload(perm(b0 + 2), w20, wsem0)
        wload(perm(b1 + 2), w21, wsem1)
        return 0

    lax.fori_loop(0, NP, pair, 0)
    swait(1, sidx1, ssem1)
    gwait(gidx0, 0, gsem0)
    wwait(perm(0), w20, wsem0)
    wwait(perm(1), w21, wsem1)
    plsc.subcore_barrier()
    pltpu.sync_copy(aggsh.at[pl.ds(s * RPT, RPT)],
                    agg_out.at[pl.ds(c * NPAD + s * RPT, RPT)])


_spmm = functools.partial(
    pl.kernel,
    out_type=jax.ShapeDtypeStruct((NC * NPAD, DH), jnp.float32),
    mesh=plsc.VectorSubcoreMesh(core_axis_name="c", subcore_axis_name="s"),
    scratch_types=[
        pltpu.VMEM((NB + 2, EB), jnp.int32),
        pltpu.VMEM((EB,), jnp.float32),
        pltpu.VMEM((EB,), jnp.float32),
        pltpu.VMEM((2, EB, DH), jnp.float32),
        pltpu.VMEM((EB,), jnp.int32),
        pltpu.VMEM((EB,), jnp.int32),
        pltpu.VMEM((EB,), jnp.int32),
        pltpu.VMEM((EB,), jnp.int32),
        pltpu.VMEM_SHARED((NPAD, DH), jnp.float32),
        pltpu.SemaphoreType.DMA,
        pltpu.SemaphoreType.DMA,
        pltpu.SemaphoreType.DMA,
        pltpu.SemaphoreType.DMA,
        pltpu.SemaphoreType.DMA,
        pltpu.SemaphoreType.DMA,
    ],
)(_spmm_body)


def _mm_body(a_ref, w_ref, alpha_ref, o_ref):
    a = a_ref[...]
    acc = jnp.dot(a[0], w_ref[0:DH, :], preferred_element_type=jnp.float32)
    acc = acc + jnp.dot(a[1], w_ref[DH:DIN, :],
                        preferred_element_type=jnp.float32)
    al = alpha_ref[0]
    o_ref[...] = jnp.maximum(acc, 0.0) + al * jnp.minimum(acc, 0.0)


_R = 1000


def _linear_prelu(agg3, W, alpha1):
    return pl.pallas_call(
        _mm_body,
        grid=(N // _R,),
        in_specs=[
            pl.BlockSpec((2, _R, DH), lambda i: (0, i, 0)),
            pl.BlockSpec((DIN, DOUT), lambda i: (0, 0)),
            pl.BlockSpec(memory_space=pltpu.SMEM),
        ],
        out_specs=pl.BlockSpec((_R, DOUT), lambda i: (i, 0)),
        out_shape=jax.ShapeDtypeStruct((N, DOUT), jnp.float32),
    )(agg3, W, alpha1)


def kernel(x, edge_index, edge_weight, W, alpha):
    src = edge_index[1].astype(jnp.int32)
    dst = edge_index[0].astype(jnp.int32)
    w = edge_weight.astype(jnp.float32)
    pad = EPAD - E
    src_p = jnp.concatenate([src, jnp.zeros((pad,), jnp.int32)])
    dst_p = jnp.concatenate([dst, jnp.zeros((pad,), jnp.int32)])
    w_p = jnp.concatenate([w, jnp.zeros((pad,), jnp.float32)])
    dhi = jnp.left_shift(dst_p, 16)
    sd = jnp.stack([dhi | src_p, dhi | (src_p + N)])
    sd4 = jnp.pad(sd.reshape(NC, NS, NB, EB), ((0, 0), (0, 0), (0, 2),
                                               (0, 0)))
    w3 = jnp.pad(w_p.reshape(NS, NB, EB), ((0, 0), (0, 2), (0, 0)))
    zrows = jnp.zeros((RPT, DH), jnp.float32)
    xcat = jnp.concatenate([x[:, :DH], x[:, DH:]], axis=0)
    agg = _spmm(xcat, sd4, w3, zrows)
    agg3 = agg.reshape(NC, NPAD, DH)
    return _linear_prelu(agg3, W, alpha.reshape(1))


# speedup vs baseline: 1.4856x; 1.0040x over previous
import functools

import jax
import jax.numpy as jnp
from jax import lax
from jax.experimental import pallas as pl
from jax.experimental.pallas import tpu as pltpu
from jax.experimental.pallas import tpu_sc as plsc

N = 10000
E = 160000
DIN = 256
DOUT = 512
DH = DIN // 2

NC = 2
NS = 16
L = 16

EB = 96
NB = 106
NP = NB // 2
ET = NB * EB
EPAD = ET * NS
NPAD = 10240
RPT = NPAD // NS

_GDN = lax.GatherDimensionNumbers(
    offset_dims=(), collapsed_slice_dims=(0,), start_index_map=(0,))


def _lane_bcast(v16, j):
    idx = jnp.full((L, 1), j, jnp.int32)
    return lax.gather(v16, idx, _GDN, slice_sizes=(1,),
                      mode=lax.GatherScatterMode.PROMISE_IN_BOUNDS)


def _spmm_body(xcat, sd4, w3, zrows, agg_out,
               sd_all, w20, w21, rows, gidx0, gidx1, sidx0, sidx1, aggsh,
               gsem0, gsem1, ssem0, ssem1, wsem0, wsem1):
    c = lax.axis_index("c")
    s = lax.axis_index("s")

    pltpu.sync_copy(sd4.at[c, s], sd_all)
    pltpu.sync_copy(zrows, aggsh.at[pl.ds(s * RPT, RPT)])
    pltpu.sync_copy(zrows.at[pl.ds(0, EB)], rows.at[1])
    plsc.subcore_barrier()

    def unpack(b, gi, si):
        for t in range(EB // L):
            wd = sd_all[b, pl.ds(t * L, L)]
            gi[pl.ds(t * L, L)] = jnp.bitwise_and(wd, 0xFFFF)
            si[pl.ds(t * L, L)] = jnp.right_shift(wd, 16)

    def scale(buf, wv):
        for g in range(EB // L):
            w16 = wv[pl.ds(g * L, L)]
            for j in range(L):
                wb = _lane_bcast(w16, j)
                e = g * L + j
                for k in range(DH // L):
                    rows[buf, e, pl.ds(k * L, L)] = (
                        rows[buf, e, pl.ds(k * L, L)] * wb)

    def wload(b, wv, sem):
        return pltpu.async_copy(w3.at[s, b], wv, sem)

    def wwait(b, wv, sem):
        pltpu.make_async_copy(w3.at[s, b], wv, sem).wait()

    EH = EB // 2

    def gather(gi, buf, sem):
        pltpu.async_copy(xcat.at[gi.at[pl.ds(0, EH)]],
                         rows.at[buf, pl.ds(0, EH)], sem)
        pltpu.async_copy(xcat.at[gi.at[pl.ds(EH, EH)]],
                         rows.at[buf, pl.ds(EH, EH)], sem)

    def gwait(gi, buf, sem):
        pltpu.make_async_copy(xcat.at[gi.at[pl.ds(0, EH)]],
                              rows.at[buf, pl.ds(0, EH)], sem).wait()
        pltpu.make_async_copy(xcat.at[gi.at[pl.ds(EH, EH)]],
                              rows.at[buf, pl.ds(EH, EH)], sem).wait()

    def scatter(buf, si, sem):
        return pltpu.async_copy(rows.at[buf], aggsh.at[si], sem, add=True)

    def swait(buf, si, sem):
        pltpu.make_async_copy(rows.at[buf], aggsh.at[si], sem).wait()

    def perm(b):
        bb = b + c * 27
        return jnp.where(bb >= NB, bb - NB, bb)

    unpack(perm(0), gidx0, sidx0)
    scatter(1, sidx0, ssem1)
    gather(gidx0, 0, gsem0)
    wload(perm(0), w20, wsem0)
    wload(perm(1), w21, wsem1)

    def pair(p, _):
        b0 = 2 * p
        b1 = b0 + 1
        swait(1, sidx1, ssem1)
        unpack(perm(b1), gidx1, sidx1)
        gather(gidx1, 1, gsem1)
        gwait(gidx0, 0, gsem0)
        wwait(perm(b0), w20, wsem0)
        scale(0, w20)
        scatter(0, sidx0, ssem0)
        gwait(gidx1, 1, gsem1)
        wwait(perm(b1), w21, wsem1)
        scale(1, w21)
        scatter(1, sidx1, ssem1)
        swait(0, sidx0, ssem0)
        unpack(perm(b0 + 2), gidx0, sidx0)
        gather(gidx0, 0, gsem0)
        wload(perm(b0 + 2), w20, wsem0)
        wload(perm(b1 + 2), w21, wsem1)
        return 0

    lax.fori_loop(0, NP, pair, 0)
    swait(1, sidx1, ssem1)
    gwait(gidx0, 0, gsem0)
    wwait(perm(0), w20, wsem0)
    wwait(perm(1), w21, wsem1)
    plsc.subcore_barrier()
    pltpu.sync_copy(aggsh.at[pl.ds(s * RPT, RPT)],
                    agg_out.at[pl.ds(c * NPAD + s * RPT, RPT)])


_spmm = functools.partial(
    pl.kernel,
    out_type=jax.ShapeDtypeStruct((NC * NPAD, DH), jnp.float32),
    mesh=plsc.VectorSubcoreMesh(core_axis_name="c", subcore_axis_name="s"),
    scratch_types=[
        pltpu.VMEM((NB + 2, EB), jnp.int32),
        pltpu.VMEM((EB,), jnp.float32),
        pltpu.VMEM((EB,), jnp.float32),
        pltpu.VMEM((2, EB, DH), jnp.float32),
        pltpu.VMEM((EB,), jnp.int32),
        pltpu.VMEM((EB,), jnp.int32),
        pltpu.VMEM((EB,), jnp.int32),
        pltpu.VMEM((EB,), jnp.int32),
        pltpu.VMEM_SHARED((NPAD, DH), jnp.float32),
        pltpu.SemaphoreType.DMA,
        pltpu.SemaphoreType.DMA,
        pltpu.SemaphoreType.DMA,
        pltpu.SemaphoreType.DMA,
        pltpu.SemaphoreType.DMA,
        pltpu.SemaphoreType.DMA,
    ],
)(_spmm_body)


def _mm_body(a_ref, w_ref, alpha_ref, o_ref):
    a = a_ref[...]
    acc = jnp.dot(a[0], w_ref[0:DH, :], preferred_element_type=jnp.float32)
    acc = acc + jnp.dot(a[1], w_ref[DH:DIN, :],
                        preferred_element_type=jnp.float32)
    al = alpha_ref[0]
    o_ref[...] = jnp.maximum(acc, 0.0) + al * jnp.minimum(acc, 0.0)


_R = 1000


def _linear_prelu(agg3, W, alpha1):
    return pl.pallas_call(
        _mm_body,
        grid=(N // _R,),
        in_specs=[
            pl.BlockSpec((2, _R, DH), lambda i: (0, i, 0)),
            pl.BlockSpec((DIN, DOUT), lambda i: (0, 0)),
            pl.BlockSpec(memory_space=pltpu.SMEM),
        ],
        out_specs=pl.BlockSpec((_R, DOUT), lambda i: (i, 0)),
        out_shape=jax.ShapeDtypeStruct((N, DOUT), jnp.float32),
    )(agg3, W, alpha1)


def kernel(x, edge_index, edge_weight, W, alpha):
    src = edge_index[1].astype(jnp.int32)
    dst = edge_index[0].astype(jnp.int32)
    w = edge_weight.astype(jnp.float32)
    pad = EPAD - E
    src_p = jnp.concatenate([src, jnp.zeros((pad,), jnp.int32)])
    dst_p = jnp.concatenate([dst, jnp.zeros((pad,), jnp.int32)])
    w_p = jnp.concatenate([w, jnp.zeros((pad,), jnp.float32)])
    dhi = jnp.left_shift(dst_p, 16)
    sd = jnp.stack([dhi | src_p, dhi | (src_p + N)])
    sd4 = jnp.pad(sd.reshape(NC, NS, NB, EB), ((0, 0), (0, 0), (0, 2),
                                               (0, 0)))
    w3 = jnp.pad(w_p.reshape(NS, NB, EB), ((0, 0), (0, 2), (0, 0)))
    zrows = jnp.zeros((RPT, DH), jnp.float32)
    xcat = jnp.concatenate([x[:, :DH], x[:, DH:]], axis=0)
    agg = _spmm(xcat, sd4, w3, zrows)
    agg3 = agg.reshape(NC, NPAD, DH)
    return _linear_prelu(agg3, W, alpha.reshape(1))
